# baseline (speedup 1.0000x reference)
import functools

import jax
import jax.numpy as jnp
from jax import lax
from jax.experimental import pallas as pl
from jax.experimental.pallas import tpu as pltpu
from jax.experimental.pallas import tpu_sc as plsc

_SMOOTHING = 0.1
_IGNORE_INDEX = 0

_N_SC = 512
_ROWS_PER_WORKER = 16
_TC_ROWS_PER_BLOCK = 32


def _loss_rows_kernel(pred_ref, tgt_ref, out_ref, *, vocab):
    x = pred_ref[...]
    t = tgt_ref[...]
    m = jnp.max(x, axis=-1, keepdims=True)
    s = jnp.sum(jnp.exp(x - m), axis=-1, keepdims=True)
    lse = m + jnp.log(s)
    sum_x = jnp.sum(x, axis=-1, keepdims=True)
    lane = jax.lax.broadcasted_iota(jnp.int32, x.shape, 1)
    pred_t = jnp.sum(jnp.where(lane == t, x, 0.0), axis=-1, keepdims=True)
    eps = _SMOOTHING / (vocab - 2)
    conf = 1.0 - _SMOOTHING
    sum_logp = sum_x - vocab * lse
    logp_t = pred_t - lse
    loss = -(eps * sum_logp + (conf - eps) * logp_t)
    out_ref[...] = jnp.where(t == _IGNORE_INDEX, 0.0, loss)


def _tc_part(pred, tgt2d, n_sc, vocab):
    n = pred.shape[0]
    r = _TC_ROWS_PER_BLOCK
    steps = (n - n_sc) // r
    off = n_sc // r
    return pl.pallas_call(
        functools.partial(_loss_rows_kernel, vocab=vocab),
        grid=(steps,),
        in_specs=[
            pl.BlockSpec((r, vocab), lambda i: (i + off, 0)),
            pl.BlockSpec((r, 1), lambda i: (i + off, 0)),
        ],
        out_specs=pl.BlockSpec((r, 1), lambda i: (i, 0)),
        out_shape=jax.ShapeDtypeStruct((n - n_sc, 1), jnp.float32),
    )(pred, tgt2d)


def _make_sc_part(vocab):
    nvec = vocab // 16
    unroll = 10
    rpw = _ROWS_PER_WORKER
    mesh = plsc.VectorSubcoreMesh(core_axis_name="c", subcore_axis_name="s")
    lanes_out = jax.ShapeDtypeStruct((_N_SC * 16,), jnp.float32)
    rows_out = jax.ShapeDtypeStruct((_N_SC,), jnp.float32)

    @functools.partial(
        pl.kernel,
        mesh=mesh,
        out_type=(lanes_out, lanes_out, lanes_out, rows_out),
        scratch_types=[
            pltpu.VMEM((vocab,), jnp.float32),
            pltpu.VMEM((rpw,), jnp.int32),
            pltpu.VMEM((rpw * 16,), jnp.float32),
            pltpu.VMEM((rpw * 16,), jnp.float32),
            pltpu.VMEM((rpw * 16,), jnp.float32),
            pltpu.VMEM((16,), jnp.float32),
            pltpu.SemaphoreType.DMA,
        ],
    )
    def sc_stats(pred_hbm, pred_flat_hbm, tgt_hbm, m_hbm, s_hbm, sx_hbm,
                 pt_hbm, row_v, tgt_v, m_st, s_st, sx_st, pt_st, sem):
        nc = 2
        wid = lax.axis_index("s") * nc + lax.axis_index("c")
        base = wid * rpw
        pltpu.sync_copy(tgt_hbm.at[pl.ds(base, rpw)], tgt_v)
        lane = lax.broadcasted_iota(jnp.int32, (16,), 0)
        zeros = jnp.zeros((16,), jnp.float32)
        t_all = tgt_v[...]
        flat_idx = (base + lane) * vocab + t_all
        pltpu.async_copy(pred_flat_hbm.at[flat_idx], pt_st, sem).wait()
        for r in range(rpw):
            pltpu.sync_copy(pred_hbm.at[base + r], row_v)

            def body1(i, m):
                for k in range(unroll):
                    m = jnp.maximum(m, row_v[pl.ds(i * (16 * unroll) + k * 16, 16)])
                return m

            m_v = lax.fori_loop(0, nvec // unroll, body1,
                                jnp.full((16,), -jnp.inf, jnp.float32))

            def body2(i, carry):
                s, sx = carry
                for k in range(unroll):
                    v = row_v[pl.ds(i * (16 * unroll) + k * 16, 16)]
                    s = s + jnp.exp(v - m_v)
                    sx = sx + v
                return s, sx

            s_v, sx_v = lax.fori_loop(0, nvec // unroll, body2, (zeros, zeros))
            m_st[pl.ds(r * 16, 16)] = m_v
            s_st[pl.ds(r * 16, 16)] = s_v
            sx_st[pl.ds(r * 16, 16)] = sx_v
        pltpu.sync_copy(m_st, m_hbm.at[pl.ds(base * 16, rpw * 16)])
        pltpu.sync_copy(s_st, s_hbm.at[pl.ds(base * 16, rpw * 16)])
        pltpu.sync_copy(sx_st, sx_hbm.at[pl.ds(base * 16, rpw * 16)])
        pltpu.sync_copy(pt_st, pt_hbm.at[pl.ds(base, rpw)])

    return sc_stats


def kernel(pred, target):
    n, vocab = pred.shape
    tgt = target.astype(jnp.int32)
    tgt2d = tgt.reshape(n, 1)
    eps = _SMOOTHING / (vocab - 2)
    conf = 1.0 - _SMOOTHING

    tc_losses = _tc_part(pred, tgt2d, _N_SC, vocab)
    m, s, sx, pt = _make_sc_part(vocab)(pred, pred.reshape(-1), tgt)

    m = m.reshape(_N_SC, 16)
    s = s.reshape(_N_SC, 16)
    sx = sx.reshape(_N_SC, 16)
    row_max = jnp.max(m, axis=1)
    row_s = jnp.sum(s * jnp.exp(m - row_max[:, None]), axis=1)
    lse = row_max + jnp.log(row_s)
    sum_x = jnp.sum(sx, axis=1)
    sc_loss = -(eps * (sum_x - vocab * lse) + (conf - eps) * (pt - lse))
    sc_loss = jnp.where(tgt[:_N_SC] == _IGNORE_INDEX, 0.0, sc_loss)
    return (jnp.sum(sc_loss) + jnp.sum(tc_losses)) / n


# speedup vs baseline: 1.0006x; 1.0006x over previous
import functools

import jax
import jax.numpy as jnp
from jax import lax
from jax.experimental import pallas as pl
from jax.experimental.pallas import tpu as pltpu
from jax.experimental.pallas import tpu_sc as plsc

_SMOOTHING = 0.1
_IGNORE_INDEX = 0

_N_SC = 512
_ROWS_PER_WORKER = 16
_TC_ROWS_PER_BLOCK = 32


def _loss_rows_kernel(pred_ref, tgt_ref, out_ref, *, vocab):
    x = pred_ref[...]
    t = tgt_ref[...]
    m = jnp.max(x, axis=-1, keepdims=True)
    s = jnp.sum(jnp.exp(x - m), axis=-1, keepdims=True)
    lse = m + jnp.log(s)
    sum_x = jnp.sum(x, axis=-1, keepdims=True)
    lane = jax.lax.broadcasted_iota(jnp.int32, x.shape, 1)
    pred_t = jnp.sum(jnp.where(lane == t, x, 0.0), axis=-1, keepdims=True)
    eps = _SMOOTHING / (vocab - 2)
    conf = 1.0 - _SMOOTHING
    sum_logp = sum_x - vocab * lse
    logp_t = pred_t - lse
    loss = -(eps * sum_logp + (conf - eps) * logp_t)
    out_ref[...] = jnp.where(t == _IGNORE_INDEX, 0.0, loss)


def _tc_part(pred, tgt2d, n_sc, vocab):
    n = pred.shape[0]
    r = _TC_ROWS_PER_BLOCK
    steps = (n - n_sc) // r
    off = n_sc // r
    return pl.pallas_call(
        functools.partial(_loss_rows_kernel, vocab=vocab),
        grid=(steps,),
        in_specs=[
            pl.BlockSpec((r, vocab), lambda i: (i + off, 0)),
            pl.BlockSpec((r, 1), lambda i: (i + off, 0)),
        ],
        out_specs=pl.BlockSpec((r, 1), lambda i: (i, 0)),
        out_shape=jax.ShapeDtypeStruct((n - n_sc, 1), jnp.float32),
    )(pred, tgt2d)


def _make_sc_part(vocab):
    nvec = vocab // 16
    unroll = 10
    rpw = _ROWS_PER_WORKER
    mesh = plsc.VectorSubcoreMesh(core_axis_name="c", subcore_axis_name="s")
    lanes_out = jax.ShapeDtypeStruct((_N_SC * 16,), jnp.float32)
    rows_out = jax.ShapeDtypeStruct((_N_SC,), jnp.float32)

    @functools.partial(
        pl.kernel,
        mesh=mesh,
        out_type=(lanes_out, lanes_out, lanes_out, rows_out),
        scratch_types=[
            pltpu.VMEM((vocab,), jnp.float32),
            pltpu.VMEM((rpw,), jnp.int32),
            pltpu.VMEM((rpw * 16,), jnp.float32),
            pltpu.VMEM((rpw * 16,), jnp.float32),
            pltpu.VMEM((rpw * 16,), jnp.float32),
            pltpu.VMEM((16,), jnp.float32),
            pltpu.SemaphoreType.DMA,
        ],
    )
    def sc_stats(pred_hbm, pred_flat_hbm, tgt_hbm, m_hbm, s_hbm, sx_hbm,
                 pt_hbm, row_v, tgt_v, m_st, s_st, sx_st, pt_st, sem):
        nc = 2
        wid = lax.axis_index("s") * nc + lax.axis_index("c")
        base = wid * rpw
        pltpu.sync_copy(tgt_hbm.at[pl.ds(base, rpw)], tgt_v)
        lane = lax.broadcasted_iota(jnp.int32, (16,), 0)
        zeros = jnp.zeros((16,), jnp.float32)
        t_all = tgt_v[...]
        flat_idx = (base + lane) * vocab + t_all
        pltpu.async_copy(pred_flat_hbm.at[flat_idx], pt_st, sem).wait()
        for r in range(rpw):
            pltpu.sync_copy(pred_hbm.at[base + r], row_v)

            def body1(i, m):
                for k in range(unroll):
                    m = jnp.maximum(m, row_v[pl.ds(i * (16 * unroll) + k * 16, 16)])
                return m

            m_v = lax.fori_loop(0, nvec // unroll, body1,
                                jnp.full((16,), -jnp.inf, jnp.float32))

            def body2(i, carry):
                s, sx = carry
                for k in range(unroll):
                    v = row_v[pl.ds(i * (16 * unroll) + k * 16, 16)]
                    s = s + jnp.exp(v - m_v)
                    sx = sx + v
                return s, sx

            s_v, sx_v = lax.fori_loop(0, nvec // unroll, body2, (zeros, zeros))
            m_st[pl.ds(r * 16, 16)] = m_v
            s_st[pl.ds(r * 16, 16)] = s_v
            sx_st[pl.ds(r * 16, 16)] = sx_v
        pltpu.sync_copy(m_st, m_hbm.at[pl.ds(base * 16, rpw * 16)])
        pltpu.sync_copy(s_st, s_hbm.at[pl.ds(base * 16, rpw * 16)])
        pltpu.sync_copy(sx_st, sx_hbm.at[pl.ds(base * 16, rpw * 16)])
        pltpu.sync_copy(pt_st, pt_hbm.at[pl.ds(base, rpw)])

    return sc_stats


def kernel(pred, target):
    n, vocab = pred.shape
    tgt = target.astype(jnp.int32)
    tgt2d = tgt.reshape(n, 1)
    eps = _SMOOTHING / (vocab - 2)
    conf = 1.0 - _SMOOTHING

    m, s, sx, pt = _make_sc_part(vocab)(pred, pred.reshape(-1), tgt)
    tc_losses = _tc_part(pred, tgt2d, _N_SC, vocab)

    m = m.reshape(_N_SC, 16)
    s = s.reshape(_N_SC, 16)
    sx = sx.reshape(_N_SC, 16)
    row_max = jnp.max(m, axis=1)
    row_s = jnp.sum(s * jnp.exp(m - row_max[:, None]), axis=1)
    lse = row_max + jnp.log(row_s)
    sum_x = jnp.sum(sx, axis=1)
    sc_loss = -(eps * (sum_x - vocab * lse) + (conf - eps) * (pt - lse))
    sc_loss = jnp.where(tgt[:_N_SC] == _IGNORE_INDEX, 0.0, sc_loss)
    return (jnp.sum(sc_loss) + jnp.sum(tc_losses)) / n


# speedup vs baseline: 1.8939x; 1.8928x over previous
import functools

import jax
import jax.numpy as jnp
from jax import lax
from jax.experimental import pallas as pl
from jax.experimental.pallas import tpu as pltpu
from jax.experimental.pallas import tpu_sc as plsc

_SMOOTHING = 0.1
_IGNORE_INDEX = 0

_N_SC = 512
_ROWS_PER_WORKER = 16
_TC_ROWS_PER_BLOCK = 32


def _loss_rows_kernel(pred_ref, tgt_ref, out_ref, *, vocab):
    x = pred_ref[...]
    t = tgt_ref[...]
    m = jnp.max(x, axis=-1, keepdims=True)
    s = jnp.sum(jnp.exp(x - m), axis=-1, keepdims=True)
    lse = m + jnp.log(s)
    sum_x = jnp.sum(x, axis=-1, keepdims=True)
    lane = jax.lax.broadcasted_iota(jnp.int32, x.shape, 1)
    pred_t = jnp.sum(jnp.where(lane == t, x, 0.0), axis=-1, keepdims=True)
    eps = _SMOOTHING / (vocab - 2)
    conf = 1.0 - _SMOOTHING
    sum_logp = sum_x - vocab * lse
    logp_t = pred_t - lse
    loss = -(eps * sum_logp + (conf - eps) * logp_t)
    out_ref[...] = jnp.where(t == _IGNORE_INDEX, 0.0, loss)


def _tc_part(pred, tgt2d, n_sc, vocab):
    n = pred.shape[0]
    r = _TC_ROWS_PER_BLOCK
    steps = (n - n_sc) // r
    off = n_sc // r
    return pl.pallas_call(
        functools.partial(_loss_rows_kernel, vocab=vocab),
        grid=(steps,),
        in_specs=[
            pl.BlockSpec((r, vocab), lambda i: (i + off, 0)),
            pl.BlockSpec((r, 1), lambda i: (i + off, 0)),
        ],
        out_specs=pl.BlockSpec((r, 1), lambda i: (i, 0)),
        out_shape=jax.ShapeDtypeStruct((n - n_sc, 1), jnp.float32),
    )(pred, tgt2d)


def _make_sc_part(vocab):
    nvec = vocab // 16
    unroll = 10
    rpw = _ROWS_PER_WORKER
    mesh = plsc.VectorSubcoreMesh(core_axis_name="c", subcore_axis_name="s")
    lanes_out = jax.ShapeDtypeStruct((_N_SC * 16,), jnp.float32)
    rows_out = jax.ShapeDtypeStruct((_N_SC,), jnp.float32)

    @functools.partial(
        pl.kernel,
        mesh=mesh,
        out_type=(lanes_out, lanes_out, lanes_out, lanes_out),
        scratch_types=[
            pltpu.VMEM((vocab,), jnp.float32),
            pltpu.VMEM((rpw,), jnp.int32),
            pltpu.VMEM((rpw * 16,), jnp.float32),
            pltpu.VMEM((rpw * 16,), jnp.float32),
            pltpu.VMEM((rpw * 16,), jnp.float32),
            pltpu.VMEM((rpw * 16,), jnp.float32),
        ],
    )
    def sc_stats(pred_hbm, tgt_hbm, m_hbm, s_hbm, sx_hbm,
                 pt_hbm, row_v, tgt_v, m_st, s_st, sx_st, pt_st):
        nc = 2
        wid = lax.axis_index("s") * nc + lax.axis_index("c")
        base = wid * rpw
        pltpu.sync_copy(tgt_hbm.at[pl.ds(base, rpw)], tgt_v)
        lane = lax.broadcasted_iota(jnp.int32, (16,), 0)
        zeros = jnp.zeros((16,), jnp.float32)
        for r in range(rpw):
            pltpu.sync_copy(pred_hbm.at[base + r], row_v)

            def body1(i, m):
                for k in range(unroll):
                    m = jnp.maximum(m, row_v[pl.ds(i * (16 * unroll) + k * 16, 16)])
                return m

            m_v = lax.fori_loop(0, nvec // unroll, body1,
                                jnp.full((16,), -jnp.inf, jnp.float32))

            def body2(i, carry):
                s, sx = carry
                for k in range(unroll):
                    v = row_v[pl.ds(i * (16 * unroll) + k * 16, 16)]
                    s = s + jnp.exp(v - m_v)
                    sx = sx + v
                return s, sx

            s_v, sx_v = lax.fori_loop(0, nvec // unroll, body2, (zeros, zeros))
            t = tgt_v[...][r]
            tb = (t // 16) * 16
            w = row_v[pl.ds(tb, 16)]
            pt_l = jnp.where(lane == t - tb, w, 0.0)
            m_st[pl.ds(r * 16, 16)] = m_v
            s_st[pl.ds(r * 16, 16)] = s_v
            sx_st[pl.ds(r * 16, 16)] = sx_v
            pt_st[pl.ds(r * 16, 16)] = pt_l
        pltpu.sync_copy(m_st, m_hbm.at[pl.ds(base * 16, rpw * 16)])
        pltpu.sync_copy(s_st, s_hbm.at[pl.ds(base * 16, rpw * 16)])
        pltpu.sync_copy(sx_st, sx_hbm.at[pl.ds(base * 16, rpw * 16)])
        pltpu.sync_copy(pt_st, pt_hbm.at[pl.ds(base * 16, rpw * 16)])

    return sc_stats


def kernel(pred, target):
    n, vocab = pred.shape
    tgt = target.astype(jnp.int32)
    tgt2d = tgt.reshape(n, 1)
    eps = _SMOOTHING / (vocab - 2)
    conf = 1.0 - _SMOOTHING

    m, s, sx, pt = _make_sc_part(vocab)(pred, tgt)
    tc_losses = _tc_part(pred, tgt2d, _N_SC, vocab)

    m = m.reshape(_N_SC, 16)
    s = s.reshape(_N_SC, 16)
    sx = sx.reshape(_N_SC, 16)
    row_max = jnp.max(m, axis=1)
    row_s = jnp.sum(s * jnp.exp(m - row_max[:, None]), axis=1)
    lse = row_max + jnp.log(row_s)
    sum_x = jnp.sum(sx, axis=1)
    pt = jnp.sum(pt.reshape(_N_SC, 16), axis=1)
    sc_loss = -(eps * (sum_x - vocab * lse) + (conf - eps) * (pt - lse))
    sc_loss = jnp.where(tgt[:_N_SC] == _IGNORE_INDEX, 0.0, sc_loss)
    return (jnp.sum(sc_loss) + jnp.sum(tc_losses)) / n


# speedup vs baseline: 2.2480x; 1.1870x over previous
import functools

import jax
import jax.numpy as jnp
from jax.experimental import pallas as pl

_SMOOTHING = 0.1
_IGNORE_INDEX = 0


def _loss_rows_kernel(pred_ref, tgt_ref, out_ref, *, vocab):
    x = pred_ref[...]
    t = tgt_ref[...]
    m = jnp.max(x, axis=-1, keepdims=True)
    s = jnp.sum(jnp.exp(x - m), axis=-1, keepdims=True)
    lse = m + jnp.log(s)
    sum_x = jnp.sum(x, axis=-1, keepdims=True)
    lane = jax.lax.broadcasted_iota(jnp.int32, x.shape, 1)
    pred_t = jnp.sum(jnp.where(lane == t, x, 0.0), axis=-1, keepdims=True)
    eps = _SMOOTHING / (vocab - 2)
    conf = 1.0 - _SMOOTHING
    sum_logp = sum_x - vocab * lse
    logp_t = pred_t - lse
    loss = -(eps * sum_logp + (conf - eps) * logp_t)
    out_ref[...] = jnp.where(t == _IGNORE_INDEX, 0.0, loss)


def kernel(pred, target):
    n, vocab = pred.shape
    rows_per_block = 32
    tgt = target.astype(jnp.int32).reshape(n, 1)
    row_losses = pl.pallas_call(
        functools.partial(_loss_rows_kernel, vocab=vocab),
        grid=(n // rows_per_block,),
        in_specs=[
            pl.BlockSpec((rows_per_block, vocab), lambda i: (i, 0)),
            pl.BlockSpec((rows_per_block, 1), lambda i: (i, 0)),
        ],
        out_specs=pl.BlockSpec((rows_per_block, 1), lambda i: (i, 0)),
        out_shape=jax.ShapeDtypeStruct((n, 1), jnp.float32),
    )(pred, tgt)
    return jnp.sum(row_losses) / n
